# Initial kernel scaffold; baseline (speedup 1.0000x reference)
import functools

import jax
import jax.numpy as jnp
from jax import lax
from jax.experimental import pallas as pl
from jax.experimental.pallas import tpu as pltpu
from jax.experimental.pallas import tpu_sc as plsc

N = 10000
E = 320000
D = 128

NC = 2
NS = 16
NW = NC * NS
E_PER_TILE = E // NW
CHUNK = 80
NCHUNK = E_PER_TILE // CHUNK
ROWS_PER_TILE = N // NS
ZROWS = 125


def _matmul_body(u_ref, w_ref, h_ref):
    h_ref[...] = jnp.dot(u_ref[...], w_ref[...],
                         preferred_element_type=jnp.float32)


def _dense_project(u_f, weight):
    BLK = 400
    return pl.pallas_call(
        _matmul_body,
        grid=(N // BLK,),
        in_specs=[pl.BlockSpec((BLK, D), lambda i: (i, 0)),
                  pl.BlockSpec((D, D), lambda i: (0, 0))],
        out_specs=pl.BlockSpec((BLK, D), lambda i: (i, 0)),
        out_shape=jax.ShapeDtypeStruct((N, D), jnp.float32),
    )(u_f, weight)


def _sc_body(h_hbm, src_hbm, dst_hbm, w_hbm, out_hbm,
             acc, sidx, didx, wbuf, rows, zbuf, gsem):
    cid = lax.axis_index("c")
    sid = lax.axis_index("s")
    wid = sid * NC + cid

    zero16 = jnp.zeros((16,), jnp.float32)

    def zero_body(i, _):
        r = i // (D // 16)
        c = (i % (D // 16)) * 16
        zbuf[r, pl.ds(c, 16)] = zero16
        return 0

    lax.fori_loop(0, ZROWS * (D // 16), zero_body, 0)

    row0 = sid * ROWS_PER_TILE
    for k in range(ROWS_PER_TILE // ZROWS):
        pltpu.sync_copy(zbuf, acc.at[pl.ds(row0 + k * ZROWS, ZROWS)])
    plsc.subcore_barrier()

    edge0 = wid * E_PER_TILE

    def chunk_body(g, _):
        base = edge0 + g * CHUNK
        pltpu.sync_copy(src_hbm.at[pl.ds(base, CHUNK)], sidx)
        pltpu.sync_copy(dst_hbm.at[pl.ds(base, CHUNK)], didx)
        pltpu.sync_copy(w_hbm.at[pl.ds(base, CHUNK)], wbuf)
        pltpu.async_copy(h_hbm.at[sidx], rows, gsem).wait()

        def edge_body(e, _):
            w = wbuf[e]
            for cb in range(D // 16):
                rows[e, pl.ds(cb * 16, 16)] = rows[e, pl.ds(cb * 16, 16)] * w
            return 0

        lax.fori_loop(0, CHUNK, edge_body, 0)
        pltpu.sync_copy(rows, acc.at[didx], add=True)
        return 0

    lax.fori_loop(0, NCHUNK, chunk_body, 0)
    plsc.subcore_barrier()

    pltpu.sync_copy(acc.at[pl.ds(row0, ROWS_PER_TILE)],
                    out_hbm.at[cid, pl.ds(row0, ROWS_PER_TILE)])


def _sc_edge_pass(h, src, dst, ew):
    mesh = plsc.VectorSubcoreMesh(core_axis_name="c", subcore_axis_name="s")
    f = pl.kernel(
        _sc_body,
        out_type=jax.ShapeDtypeStruct((NC, N, D), jnp.float32),
        mesh=mesh,
        scratch_types=[
            pltpu.VMEM_SHARED((N, D), jnp.float32),
            pltpu.VMEM((CHUNK,), jnp.int32),
            pltpu.VMEM((CHUNK,), jnp.int32),
            pltpu.VMEM((CHUNK,), jnp.float32),
            pltpu.VMEM((CHUNK, D), jnp.float32),
            pltpu.VMEM((ZROWS, D), jnp.float32),
            pltpu.SemaphoreType.DMA,
        ],
    )
    return f(h, src, dst, ew)


def _add_body(p_ref, o_ref):
    o_ref[...] = p_ref[0] + p_ref[1]


def _final_add(partials):
    BLK = 400
    return pl.pallas_call(
        _add_body,
        grid=(N // BLK,),
        in_specs=[pl.BlockSpec((NC, BLK, D), lambda i: (0, i, 0))],
        out_specs=pl.BlockSpec((BLK, D), lambda i: (i, 0)),
        out_shape=jax.ShapeDtypeStruct((N, D), jnp.float32),
    )(partials)


def kernel(u_f, edge_index, edge_weight, weight):
    h = _dense_project(u_f, weight)
    dst = edge_index[0].astype(jnp.int32)
    src = edge_index[1].astype(jnp.int32)
    partials = _sc_edge_pass(h, src, dst, edge_weight)
    return _final_add(partials)


# speedup vs baseline: 3.8980x; 3.8980x over previous
import functools

import jax
import jax.numpy as jnp
from jax import lax
from jax.experimental import pallas as pl
from jax.experimental.pallas import tpu as pltpu
from jax.experimental.pallas import tpu_sc as plsc

N = 10000
E = 320000
D = 128

NC = 2
NS = 16
NW = NC * NS
E_PER_TILE = E // NW
CHUNK = 80
NCHUNK = E_PER_TILE // CHUNK
NP = 10240
ROWS_PER_TILE = NP // NS
ZROWS = 128


def _matmul_body(u_ref, w_ref, h_ref):
    h_ref[...] = jnp.dot(u_ref[...], w_ref[...],
                         preferred_element_type=jnp.float32)


def _dense_project(u_f, weight):
    BLK = 400
    return pl.pallas_call(
        _matmul_body,
        grid=(N // BLK,),
        in_specs=[pl.BlockSpec((BLK, D), lambda i: (i, 0)),
                  pl.BlockSpec((D, D), lambda i: (0, 0))],
        out_specs=pl.BlockSpec((BLK, D), lambda i: (i, 0)),
        out_shape=jax.ShapeDtypeStruct((N, D), jnp.float32),
    )(u_f, weight)


def _sc_body(h_hbm, src_hbm, dst_hbm, w_hbm, out_hbm,
             acc, sidx, didx, wbuf, rows, zbuf, gsem):
    cid = lax.axis_index("c")
    sid = lax.axis_index("s")
    wid = sid * NC + cid

    zero16 = jnp.zeros((16,), jnp.float32)

    def zero_body(i, _):
        r = i // (D // 16)
        c = (i % (D // 16)) * 16
        zbuf[r, pl.ds(c, 16)] = zero16
        return 0

    lax.fori_loop(0, ZROWS * (D // 16), zero_body, 0)

    row0 = sid * ROWS_PER_TILE
    for k in range(ROWS_PER_TILE // ZROWS):
        pltpu.sync_copy(zbuf, acc.at[pl.ds(row0 + k * ZROWS, ZROWS)])
    plsc.subcore_barrier()

    edge0 = wid * E_PER_TILE

    def chunk_body(g, _):
        base = edge0 + g * CHUNK
        pltpu.sync_copy(src_hbm.at[pl.ds(base, CHUNK)], sidx)
        pltpu.sync_copy(dst_hbm.at[pl.ds(base, CHUNK)], didx)
        pltpu.sync_copy(w_hbm.at[pl.ds(base, CHUNK)], wbuf)
        pltpu.async_copy(h_hbm.at[sidx], rows, gsem).wait()

        def grp_body(g16, _):
            e0 = g16 * 16
            wv = wbuf[pl.ds(e0, 16)]
            for j in range(16):
                w = wv[j]
                e = e0 + j
                for cb in range(D // 16):
                    rows[e, pl.ds(cb * 16, 16)] = rows[e, pl.ds(cb * 16, 16)] * w
            return 0

        lax.fori_loop(0, CHUNK // 16, grp_body, 0)
        pltpu.sync_copy(rows, acc.at[didx], add=True)
        return 0

    lax.fori_loop(0, NCHUNK, chunk_body, 0)
    plsc.subcore_barrier()

    pltpu.sync_copy(acc.at[pl.ds(row0, ROWS_PER_TILE)],
                    out_hbm.at[cid, pl.ds(row0, ROWS_PER_TILE)])


def _sc_edge_pass(h, src, dst, ew):
    mesh = plsc.VectorSubcoreMesh(core_axis_name="c", subcore_axis_name="s")
    f = pl.kernel(
        _sc_body,
        out_type=jax.ShapeDtypeStruct((NC, NP, D), jnp.float32),
        mesh=mesh,
        scratch_types=[
            pltpu.VMEM_SHARED((NP, D), jnp.float32),
            pltpu.VMEM((CHUNK,), jnp.int32),
            pltpu.VMEM((CHUNK,), jnp.int32),
            pltpu.VMEM((CHUNK,), jnp.float32),
            pltpu.VMEM((CHUNK, D), jnp.float32),
            pltpu.VMEM((ZROWS, D), jnp.float32),
            pltpu.SemaphoreType.DMA,
        ],
    )
    return f(h, src, dst, ew)


def _add_body(p_ref, o_ref):
    o_ref[...] = p_ref[0] + p_ref[1]


def _final_add(partials):
    BLK = 400
    return pl.pallas_call(
        _add_body,
        grid=(N // BLK,),
        in_specs=[pl.BlockSpec((NC, BLK, D), lambda i: (0, i, 0))],
        out_specs=pl.BlockSpec((BLK, D), lambda i: (i, 0)),
        out_shape=jax.ShapeDtypeStruct((N, D), jnp.float32),
    )(partials)


def kernel(u_f, edge_index, edge_weight, weight):
    h = _dense_project(u_f, weight)
    dst = edge_index[0].astype(jnp.int32)
    src = edge_index[1].astype(jnp.int32)
    partials = _sc_edge_pass(h, src, dst, edge_weight)
    return _final_add(partials[:, :N])
